# baseline (speedup 1.0000x reference)
import functools

import numpy as np
import jax
import jax.numpy as jnp
from jax.experimental import pallas as pl
from jax.experimental.pallas import tpu as pltpu

BATCH = 1024
CONV_LENGTH = 49
D_MODEL = 768
ROWS = BATCH * CONV_LENGTH
BLOCK_ROWS = 512
GRID = ROWS // BLOCK_ROWS


def _mask_np() -> np.ndarray:
    import math

    n = BATCH * CONV_LENGTH
    x0 = np.zeros(n, dtype=np.uint32)
    x1 = np.arange(n, dtype=np.uint32)
    ks = [np.uint32(0), np.uint32(0), np.uint32(0x1BD11BDA)]
    rotations = [(13, 15, 26, 6), (17, 29, 16, 24)]

    def rotl(v, r):
        return (v << np.uint32(r)) | (v >> np.uint32(32 - r))

    x0 = x0 + ks[0]
    x1 = x1 + ks[1]
    for i in range(5):
        for r in rotations[i % 2]:
            x0 = x0 + x1
            x1 = rotl(x1, r)
            x1 = x1 ^ x0
        x0 = x0 + ks[(i + 1) % 3]
        x1 = x1 + ks[(i + 2) % 3] + np.uint32(i + 1)

    bits = x0 ^ x1
    u01 = ((bits >> np.uint32(9)) | np.uint32(0x3F800000)).view(
        np.float32) - np.float32(1.0)
    lo = np.float32(np.nextafter(np.float32(-1.0), np.float32(0.0)))
    hi = np.float32(1.0)
    val = np.maximum(lo, u01 * (hi - lo) + lo)
    thresh = math.erf(0.5 / math.sqrt(2.0))
    return (val.astype(np.float64) > thresh).reshape(BATCH, CONV_LENGTH)


_MASK_NP = _mask_np()


def _select_body(m_ref, e_ref, x_ref, o_ref):
    o_ref[...] = jnp.where(m_ref[...] != 0, e_ref[...], x_ref[...])


def _mask_copy_body(mi_ref, mo_ref):
    mo_ref[...] = mi_ref[...]


def kernel(x, mask_embedding):
    mask_np = _MASK_NP
    mcol = jnp.asarray(mask_np.reshape(ROWS, 1).astype(np.float32))
    mask2d = jnp.asarray(mask_np.reshape(392, 128))

    xr = x.reshape(ROWS, D_MODEL)
    emb = mask_embedding.reshape(1, D_MODEL)

    out = pl.pallas_call(
        _select_body,
        grid=(GRID,),
        in_specs=[
            pl.BlockSpec((BLOCK_ROWS, 1), lambda i: (i, 0)),
            pl.BlockSpec((1, D_MODEL), lambda i: (0, 0)),
            pl.BlockSpec((BLOCK_ROWS, D_MODEL), lambda i: (i, 0)),
        ],
        out_specs=pl.BlockSpec((BLOCK_ROWS, D_MODEL), lambda i: (i, 0)),
        out_shape=jax.ShapeDtypeStruct((ROWS, D_MODEL), x.dtype),
        compiler_params=pltpu.CompilerParams(
            dimension_semantics=("parallel",)),
    )(mcol, emb, xr)

    mask_out = pl.pallas_call(
        _mask_copy_body,
        out_shape=jax.ShapeDtypeStruct((392, 128), jnp.bool_),
    )(mask2d)

    return out.reshape(BATCH, CONV_LENGTH, D_MODEL), mask_out.reshape(
        BATCH, CONV_LENGTH)


# speedup vs baseline: 1.8490x; 1.8490x over previous
import functools

import numpy as np
import jax
import jax.numpy as jnp
from jax.experimental import pallas as pl
from jax.experimental.pallas import tpu as pltpu

BATCH = 1024
CONV_LENGTH = 49
D_MODEL = 768
ROWS = BATCH * CONV_LENGTH
BLOCK_ROWS = 512
GRID = ROWS // BLOCK_ROWS


def _mask_np() -> np.ndarray:
    import math

    n = BATCH * CONV_LENGTH
    x0 = np.zeros(n, dtype=np.uint32)
    x1 = np.arange(n, dtype=np.uint32)
    ks = [np.uint32(0), np.uint32(0), np.uint32(0x1BD11BDA)]
    rotations = [(13, 15, 26, 6), (17, 29, 16, 24)]

    def rotl(v, r):
        return (v << np.uint32(r)) | (v >> np.uint32(32 - r))

    x0 = x0 + ks[0]
    x1 = x1 + ks[1]
    for i in range(5):
        for r in rotations[i % 2]:
            x0 = x0 + x1
            x1 = rotl(x1, r)
            x1 = x1 ^ x0
        x0 = x0 + ks[(i + 1) % 3]
        x1 = x1 + ks[(i + 2) % 3] + np.uint32(i + 1)

    bits = x0 ^ x1
    u01 = ((bits >> np.uint32(9)) | np.uint32(0x3F800000)).view(
        np.float32) - np.float32(1.0)
    lo = np.float32(np.nextafter(np.float32(-1.0), np.float32(0.0)))
    hi = np.float32(1.0)
    val = np.maximum(lo, u01 * (hi - lo) + lo)
    thresh = math.erf(0.5 / math.sqrt(2.0))
    return (val.astype(np.float64) > thresh).reshape(BATCH, CONV_LENGTH)


_MASK_NP = _mask_np()


BLOCK_B = 8


def _select_body(m_ref, e_ref, x_ref, o_ref, om_ref):
    m = m_ref[...]
    o_ref[...] = jnp.where(m[:, :, None] != 0, e_ref[...], x_ref[...])
    om_ref[...] = m != 0


def kernel(x, mask_embedding):
    mask_f32 = jnp.asarray(_MASK_NP.astype(np.float32))
    emb = mask_embedding.reshape(1, 1, D_MODEL)

    out, mask_out = pl.pallas_call(
        _select_body,
        grid=(BATCH // BLOCK_B,),
        in_specs=[
            pl.BlockSpec((BLOCK_B, CONV_LENGTH), lambda i: (i, 0)),
            pl.BlockSpec((1, 1, D_MODEL), lambda i: (0, 0, 0)),
            pl.BlockSpec((BLOCK_B, CONV_LENGTH, D_MODEL), lambda i: (i, 0, 0)),
        ],
        out_specs=[
            pl.BlockSpec((BLOCK_B, CONV_LENGTH, D_MODEL), lambda i: (i, 0, 0)),
            pl.BlockSpec((BLOCK_B, CONV_LENGTH), lambda i: (i, 0)),
        ],
        out_shape=[
            jax.ShapeDtypeStruct((BATCH, CONV_LENGTH, D_MODEL), x.dtype),
            jax.ShapeDtypeStruct((BATCH, CONV_LENGTH), jnp.bool_),
        ],
        compiler_params=pltpu.CompilerParams(
            dimension_semantics=("parallel",)),
    )(mask_f32, emb, x)

    return out, mask_out


# speedup vs baseline: 2.0452x; 1.1061x over previous
import functools

import numpy as np
import jax
import jax.numpy as jnp
from jax.experimental import pallas as pl
from jax.experimental.pallas import tpu as pltpu

BATCH = 1024
CONV_LENGTH = 49
D_MODEL = 768
ROWS = BATCH * CONV_LENGTH
BLOCK_ROWS = 512
GRID = ROWS // BLOCK_ROWS


def _mask_np() -> np.ndarray:
    import math

    n = BATCH * CONV_LENGTH
    x0 = np.zeros(n, dtype=np.uint32)
    x1 = np.arange(n, dtype=np.uint32)
    ks = [np.uint32(0), np.uint32(0), np.uint32(0x1BD11BDA)]
    rotations = [(13, 15, 26, 6), (17, 29, 16, 24)]

    def rotl(v, r):
        return (v << np.uint32(r)) | (v >> np.uint32(32 - r))

    x0 = x0 + ks[0]
    x1 = x1 + ks[1]
    for i in range(5):
        for r in rotations[i % 2]:
            x0 = x0 + x1
            x1 = rotl(x1, r)
            x1 = x1 ^ x0
        x0 = x0 + ks[(i + 1) % 3]
        x1 = x1 + ks[(i + 2) % 3] + np.uint32(i + 1)

    bits = x0 ^ x1
    u01 = ((bits >> np.uint32(9)) | np.uint32(0x3F800000)).view(
        np.float32) - np.float32(1.0)
    lo = np.float32(np.nextafter(np.float32(-1.0), np.float32(0.0)))
    hi = np.float32(1.0)
    val = np.maximum(lo, u01 * (hi - lo) + lo)
    thresh = math.erf(0.5 / math.sqrt(2.0))
    return (val.astype(np.float64) > thresh).reshape(BATCH, CONV_LENGTH)


_MASK_NP = _mask_np()


BLOCK_B = 32


def _select_body(m_ref, e_ref, x_ref, o_ref):
    i = pl.program_id(0)
    m = m_ref[pl.ds(i * BLOCK_B, BLOCK_B), :]
    o_ref[...] = jnp.where(m[:, :, None] != 0, e_ref[...], x_ref[...])


def _mask_body(m_ref, om_ref):
    om_ref[...] = m_ref[...] != 0


def kernel(x, mask_embedding):
    mask_f32 = jnp.asarray(_MASK_NP.astype(np.float32))
    emb = mask_embedding.reshape(1, 1, D_MODEL)

    out = pl.pallas_call(
        _select_body,
        grid=(BATCH // BLOCK_B,),
        in_specs=[
            pl.BlockSpec((BATCH, CONV_LENGTH), lambda i: (0, 0)),
            pl.BlockSpec((1, 1, D_MODEL), lambda i: (0, 0, 0)),
            pl.BlockSpec((BLOCK_B, CONV_LENGTH, D_MODEL), lambda i: (i, 0, 0)),
        ],
        out_specs=pl.BlockSpec(
            (BLOCK_B, CONV_LENGTH, D_MODEL), lambda i: (i, 0, 0)),
        out_shape=jax.ShapeDtypeStruct(
            (BATCH, CONV_LENGTH, D_MODEL), x.dtype),
        compiler_params=pltpu.CompilerParams(
            dimension_semantics=("arbitrary",)),
    )(mask_f32, emb, x)

    mask_out = pl.pallas_call(
        _mask_body,
        out_shape=jax.ShapeDtypeStruct((BATCH, CONV_LENGTH), jnp.bool_),
    )(mask_f32)

    return out, mask_out


# speedup vs baseline: 2.1260x; 1.0395x over previous
import numpy as np
import jax
import jax.numpy as jnp
from jax.experimental import pallas as pl
from jax.experimental.pallas import tpu as pltpu

BATCH = 1024
CONV_LENGTH = 49
D_MODEL = 768
TA = 48


def _mask_np() -> np.ndarray:
    import math

    n = BATCH * CONV_LENGTH
    x0 = np.zeros(n, dtype=np.uint32)
    x1 = np.arange(n, dtype=np.uint32)
    ks = [np.uint32(0), np.uint32(0), np.uint32(0x1BD11BDA)]
    rotations = [(13, 15, 26, 6), (17, 29, 16, 24)]

    def rotl(v, r):
        return (v << np.uint32(r)) | (v >> np.uint32(32 - r))

    x0 = x0 + ks[0]
    x1 = x1 + ks[1]
    for i in range(5):
        for r in rotations[i % 2]:
            x0 = x0 + x1
            x1 = rotl(x1, r)
            x1 = x1 ^ x0
        x0 = x0 + ks[(i + 1) % 3]
        x1 = x1 + ks[(i + 2) % 3] + np.uint32(i + 1)

    bits = x0 ^ x1
    u01 = ((bits >> np.uint32(9)) | np.uint32(0x3F800000)).view(
        np.float32) - np.float32(1.0)
    lo = np.float32(np.nextafter(np.float32(-1.0), np.float32(0.0)))
    hi = np.float32(1.0)
    val = np.maximum(lo, u01 * (hi - lo) + lo)
    thresh = math.erf(0.5 / math.sqrt(2.0))
    return (val.astype(np.float64) > thresh).reshape(BATCH, CONV_LENGTH)


_MASK_NP = _mask_np()

CHUNK = 32
NCH = BATCH // CHUNK
NB = 3


def _select_body(m_any, e_any, x_ref, o_ref, *scratch):
    bufsA = scratch[:NB]
    bufsB = scratch[NB:2 * NB]
    m_v = scratch[2 * NB]
    e_v = scratch[2 * NB + 1]
    msem = scratch[2 * NB + 2]
    rA = scratch[2 * NB + 3:2 * NB + 3 + NB]
    rB = scratch[2 * NB + 3 + NB:2 * NB + 3 + 2 * NB]
    wA = scratch[2 * NB + 3 + 2 * NB:2 * NB + 3 + 3 * NB]
    wB = scratch[2 * NB + 3 + 3 * NB:2 * NB + 3 + 4 * NB]

    cm = pltpu.make_async_copy(m_any, m_v, msem)
    cm.start()
    ce = pltpu.make_async_copy(e_any, e_v, msem)
    ce.start()
    cm.wait()
    ce.wait()

    def rdA(i):
        p = i % NB
        return pltpu.make_async_copy(
            x_ref.at[pl.ds(i * CHUNK, CHUNK), pl.ds(0, TA)], bufsA[p], rA[p])

    def rdB(i):
        p = i % NB
        return pltpu.make_async_copy(
            x_ref.at[pl.ds(i * CHUNK, CHUNK), pl.ds(TA, 1)], bufsB[p], rB[p])

    def wrA(i):
        p = i % NB
        return pltpu.make_async_copy(
            bufsA[p], o_ref.at[pl.ds(i * CHUNK, CHUNK), pl.ds(0, TA)], wA[p])

    def wrB(i):
        p = i % NB
        return pltpu.make_async_copy(
            bufsB[p], o_ref.at[pl.ds(i * CHUNK, CHUNK), pl.ds(TA, 1)], wB[p])

    def select(i):
        p = i % NB
        m = m_v[pl.ds(i * CHUNK, CHUNK), :]
        e = e_v[...][None]
        bufsA[p][...] = jnp.where(
            m[:, :TA, None] != 0, e, bufsA[p][...])
        bufsB[p][...] = jnp.where(
            m[:, TA:, None] != 0, e, bufsB[p][...])

    for j in range(NB - 1):
        rdA(j).start()
        rdB(j).start()
    waited = set()
    for i in range(NCH):
        rdA(i).wait()
        rdB(i).wait()
        select(i)
        wrA(i).start()
        wrB(i).start()
        nxt = i + NB - 1
        if nxt < NCH:
            if i >= 1:
                wrA(i - 1).wait()
                wrB(i - 1).wait()
                waited.add(i - 1)
            rdA(nxt).start()
            rdB(nxt).start()
    for j in range(NCH):
        if j not in waited:
            wrA(j).wait()
            wrB(j).wait()


def _mask_body(m_ref, om_ref):
    om_ref[...] = m_ref[...] != 0


def kernel(x, mask_embedding):
    mask_f32 = jnp.asarray(_MASK_NP.astype(np.float32))
    emb = mask_embedding.reshape(1, D_MODEL)

    out = pl.pallas_call(
        _select_body,
        in_specs=[
            pl.BlockSpec(memory_space=pl.ANY),
            pl.BlockSpec(memory_space=pl.ANY),
            pl.BlockSpec(memory_space=pl.ANY),
        ],
        out_specs=pl.BlockSpec(memory_space=pl.ANY),
        out_shape=jax.ShapeDtypeStruct((BATCH, CONV_LENGTH, D_MODEL), x.dtype),
        scratch_shapes=(
            [pltpu.VMEM((CHUNK, TA, D_MODEL), jnp.float32)] * NB
            + [pltpu.VMEM((CHUNK, 1, D_MODEL), jnp.float32)] * NB
            + [pltpu.VMEM((BATCH, CONV_LENGTH), jnp.float32),
               pltpu.VMEM((1, D_MODEL), jnp.float32),
               pltpu.SemaphoreType.DMA]
            + [pltpu.SemaphoreType.DMA] * (4 * NB)),
    )(mask_f32, emb, x)

    mask_out = pl.pallas_call(
        _mask_body,
        out_shape=jax.ShapeDtypeStruct((BATCH, CONV_LENGTH), jnp.bool_),
    )(mask_f32)

    return out, mask_out
